# baseline (speedup 1.0000x reference)
import functools
import math

import jax
import jax.numpy as jnp
from jax import lax
from jax.experimental import pallas as pl
from jax.experimental.pallas import tpu as pltpu
from jax.experimental.pallas import tpu_sc as plsc

N_NODES = 10000
N_EDGES = 320000
G = 200
S = N_NODES // G
D = 128
NUM_LAYERS = 3
RATIO = 0.8

NC, NS = 2, 16
NW = NC * NS
EPT = N_EDGES // NW
ECH = 80
NCH = EPT // ECH
N_PAD = 10240
ROWS_A = N_PAD // NS
ZROWS = 160

@functools.cache
def _sc_mesh():
    return plsc.VectorSubcoreMesh(core_axis_name="c", subcore_axis_name="s")


def _wid():
    return lax.axis_index("s") * NC + lax.axis_index("c")


def _embed_body(emb_hbm, xidx_hbm, h0_hbm, idx_v, rows_v, sem):
    w = _wid()
    for t in range(4):
        ch = w + t * NW

        @pl.when(ch < N_NODES // ECH)
        def _():
            base = ch * ECH
            pltpu.sync_copy(xidx_hbm.at[pl.ds(base, ECH)], idx_v)
            pltpu.async_copy(emb_hbm.at[idx_v], rows_v, sem).wait()
            pltpu.sync_copy(rows_v, h0_hbm.at[pl.ds(base, ECH)])


@functools.cache
def _embed_call():
    return pl.kernel(
        _embed_body,
        out_type=jax.ShapeDtypeStruct((N_NODES, D), jnp.float32),
        mesh=_sc_mesh(),
        scratch_types=[
            pltpu.VMEM((ECH,), jnp.int32),
            pltpu.VMEM((ECH, D), jnp.float32),
            pltpu.SemaphoreType.DMA,
        ],
    )


def _edge_agg_body(h_hbm, alive_hbm, src_hbm, dst_hbm, za_hbm, zc_hbm,
                   pagg_hbm, pcnt_hbm,
                   agg_sh, cnt_sh, zbuf_v, src_v, dst_v, rows_v, av_v,
                   sem, sem2):
    c = lax.axis_index("c")
    s = lax.axis_index("s")
    w = s * NC + c

    pltpu.sync_copy(za_hbm, zbuf_v)
    for r in range(ROWS_A // ZROWS):
        pltpu.sync_copy(zbuf_v,
                        agg_sh.at[pl.ds(s * ROWS_A + r * ZROWS, ZROWS)])
    pltpu.sync_copy(zc_hbm, cnt_sh.at[pl.ds(s * ROWS_A, ROWS_A)])
    plsc.subcore_barrier()

    def body(i, carry):
        base = w * EPT + i * ECH
        d1 = pltpu.async_copy(src_hbm.at[pl.ds(base, ECH)], src_v, sem)
        d2 = pltpu.async_copy(dst_hbm.at[pl.ds(base, ECH)], dst_v, sem2)
        d1.wait()
        d2.wait()
        g1 = pltpu.async_copy(h_hbm.at[src_v], rows_v, sem)
        g2 = pltpu.async_copy(alive_hbm.at[src_v], av_v, sem2)
        g1.wait()
        g2.wait()
        s1 = pltpu.async_copy(rows_v, agg_sh.at[dst_v], sem, add=True)
        s2 = pltpu.async_copy(av_v, cnt_sh.at[dst_v], sem2, add=True)
        s1.wait()
        s2.wait()
        return carry

    lax.fori_loop(0, NCH, body, 0)
    plsc.subcore_barrier()

    pltpu.sync_copy(agg_sh.at[pl.ds(s * ROWS_A, ROWS_A)],
                    pagg_hbm.at[c, pl.ds(s * ROWS_A, ROWS_A)])
    pltpu.sync_copy(cnt_sh.at[pl.ds(s * ROWS_A, ROWS_A)],
                    pcnt_hbm.at[c, pl.ds(s * ROWS_A, ROWS_A)])


@functools.cache
def _edge_agg_call():
    return pl.kernel(
        _edge_agg_body,
        out_type=[jax.ShapeDtypeStruct((NC, N_PAD, D), jnp.float32),
                  jax.ShapeDtypeStruct((NC, N_PAD), jnp.float32)],
        mesh=_sc_mesh(),
        scratch_types=[
            pltpu.VMEM_SHARED((N_PAD, D), jnp.float32),
            pltpu.VMEM_SHARED((N_PAD,), jnp.float32),
            pltpu.VMEM((ZROWS, D), jnp.float32),
            pltpu.VMEM((ECH,), jnp.int32),
            pltpu.VMEM((ECH,), jnp.int32),
            pltpu.VMEM((ECH, D), jnp.float32),
            pltpu.VMEM((ECH,), jnp.float32),
            pltpu.SemaphoreType.DMA,
            pltpu.SemaphoreType.DMA,
        ],
    )


NEG = -3e38
BR1 = 2000
GB = 40
BR2 = GB * S


def _tanh(x):
    e = jnp.exp(2.0 * jnp.clip(x, -15.0, 15.0))
    return (e - 1.0) / (e + 1.0)


def _dotd(a, b):
    return jnp.dot(a.astype(jnp.bfloat16), b.astype(jnp.bfloat16),
                   preferred_element_type=jnp.float32)


def _k1_body(pagg_ref, pcnt_ref, h_ref, alive_ref, wl_ref, bl_ref, wr_ref,
             om_ref, stats_ref):
    agg = pagg_ref[0] + pagg_ref[1]
    cnt = pcnt_ref[0] + pcnt_ref[1]
    mean = agg / jnp.maximum(cnt, 1.0)
    out = _dotd(mean, wl_ref[...]) + bl_ref[...] + _dotd(h_ref[...], wr_ref[...])
    nrm = jnp.sqrt(jnp.sum(out * out, axis=1, keepdims=True))
    out = out / jnp.maximum(nrm, 1e-12)
    om = out * alive_ref[...]
    om_ref[...] = om
    st = jnp.concatenate([jnp.sum(om, axis=0, keepdims=True),
                          jnp.sum(om * om, axis=0, keepdims=True)], axis=0)

    @pl.when(pl.program_id(0) == 0)
    def _():
        stats_ref[...] = st

    @pl.when(pl.program_id(0) != 0)
    def _():
        stats_ref[...] += st


def _k1_call():
    nb = N_NODES // BR1
    return pl.pallas_call(
        _k1_body,
        grid=(nb,),
        in_specs=[
            pl.BlockSpec((NC, BR1, D), lambda i: (0, i, 0)),
            pl.BlockSpec((NC, BR1, 1), lambda i: (0, i, 0)),
            pl.BlockSpec((BR1, D), lambda i: (i, 0)),
            pl.BlockSpec((BR1, 1), lambda i: (i, 0)),
            pl.BlockSpec((D, D), lambda i: (0, 0)),
            pl.BlockSpec((1, D), lambda i: (0, 0)),
            pl.BlockSpec((D, D), lambda i: (0, 0)),
        ],
        out_specs=[
            pl.BlockSpec((BR1, D), lambda i: (i, 0)),
            pl.BlockSpec((2, D), lambda i: (0, 0)),
        ],
        out_shape=[jax.ShapeDtypeStruct((N_NODES, D), jnp.float32),
                   jax.ShapeDtypeStruct((2, D), jnp.float32)],
    )


def _k2_body(om_ref, stats_ref, alive_ref, bng_ref, bnb_ref, pw_ref,
             hout_ref, aout_ref, feat_ref, *, n_in, k):
    n_cur = float(G * n_in)
    mu = stats_ref[0:1, :] / n_cur
    var = stats_ref[1:2, :] / n_cur - mu * mu
    alive3 = alive_ref[...]
    alive2 = alive3.reshape(BR2, 1)
    out = (om_ref[...] - mu) / jnp.sqrt(var + 1e-5) * bng_ref[...] + bnb_ref[...]
    out = jnp.maximum(out, 0.0) * alive2

    pw = pw_ref[...]
    wn = jnp.sqrt(jnp.sum(pw * pw))
    score2 = _dotd(out, pw) / wn
    score3 = jnp.where(alive3 > 0, score2.reshape(GB, S, 1), jnp.float32(NEG))

    slot = lax.broadcasted_iota(jnp.int32, (GB, S, 1), 1)
    rank = jnp.zeros((GB, S, 1), jnp.float32)
    for l in range(S):
        sl = score3[:, l:l + 1, :]
        rank = rank + jnp.where(sl > score3, 1.0, 0.0)
        rank = rank + jnp.where((sl == score3) & (l < slot), 1.0, 0.0)
    keep3 = jnp.where((rank < k) & (alive3 > 0), 1.0, 0.0)

    h3 = out.reshape(GB, S, D) * (_tanh(score3) * keep3)
    hout_ref[...] = h3.reshape(BR2, D)
    aout_ref[...] = keep3
    hm = jnp.sum(h3, axis=1) / float(k)
    hx = jnp.max(jnp.where(keep3 > 0, h3, jnp.float32(NEG)), axis=1)
    feat_ref[...] = jnp.concatenate([hm, hx], axis=1)


def _k2_call(n_in, k):
    nb = G // GB
    return pl.pallas_call(
        functools.partial(_k2_body, n_in=n_in, k=k),
        grid=(nb,),
        in_specs=[
            pl.BlockSpec((BR2, D), lambda i: (i, 0)),
            pl.BlockSpec((2, D), lambda i: (0, 0)),
            pl.BlockSpec((GB, S, 1), lambda i: (i, 0, 0)),
            pl.BlockSpec((1, D), lambda i: (0, 0)),
            pl.BlockSpec((1, D), lambda i: (0, 0)),
            pl.BlockSpec((D, 1), lambda i: (0, 0)),
        ],
        out_specs=[
            pl.BlockSpec((BR2, D), lambda i: (i, 0)),
            pl.BlockSpec((GB, S, 1), lambda i: (i, 0, 0)),
            pl.BlockSpec((GB, 2 * D), lambda i: (i, 0)),
        ],
        out_shape=[jax.ShapeDtypeStruct((N_NODES, D), jnp.float32),
                   jax.ShapeDtypeStruct((G, S, 1), jnp.float32),
                   jax.ShapeDtypeStruct((G, 2 * D), jnp.float32)],
    )


def _mlp_body(z_ref, w1_ref, b1_ref, g1_ref, be1_ref, w2_ref, b2_ref,
              g2_ref, be2_ref, w3_ref, b3_ref, o_ref):
    z = jnp.maximum(_dotd(z_ref[...], w1_ref[...]) + b1_ref[...], 0.0)
    mu = jnp.sum(z, axis=0, keepdims=True) / float(G)
    var = jnp.sum((z - mu) ** 2, axis=0, keepdims=True) / float(G)
    z = (z - mu) / jnp.sqrt(var + 1e-5) * g1_ref[...] + be1_ref[...]
    z = jnp.maximum(_dotd(z, w2_ref[...]) + b2_ref[...], 0.0)
    mu = jnp.sum(z, axis=0, keepdims=True) / float(G)
    var = jnp.sum((z - mu) ** 2, axis=0, keepdims=True) / float(G)
    z = (z - mu) / jnp.sqrt(var + 1e-5) * g2_ref[...] + be2_ref[...]
    z = _dotd(z, w3_ref[...]) + b3_ref[...]
    o_ref[...] = 1.0 / (1.0 + jnp.exp(-z))


_mlp_call = pl.pallas_call(
    _mlp_body,
    out_shape=jax.ShapeDtypeStruct((G, 1), jnp.float32),
)


def kernel(x, edge_index, batch, emb, Wl, bl, Wr, bn_g, bn_b, pool_w,
           W1, b1, g1, beta1, W2, b2, g2, beta2, W3, b3):
    del batch
    f32 = jnp.float32
    xidx = x.reshape(-1).astype(jnp.int32)
    src = edge_index[0].astype(jnp.int32)
    dst = edge_index[1].astype(jnp.int32)
    za = jnp.zeros((ZROWS, D), f32)
    zc = jnp.zeros((ROWS_A,), f32)

    h = _embed_call()(emb.astype(f32), xidx)
    alive3 = jnp.ones((G, S, 1), f32)

    n_in = S
    feats = []
    for i in range(NUM_LAYERS):
        k = math.ceil(RATIO * n_in)
        pagg, pcnt = _edge_agg_call()(h, alive3.reshape(N_NODES), src, dst,
                                      za, zc)
        pagg = pagg[:, :N_NODES]
        pcnt = pcnt[:, :N_NODES].reshape(NC, N_NODES, 1)
        om, stats = _k1_call()(pagg, pcnt, h, alive3.reshape(N_NODES, 1),
                               Wl[i], bl[i].reshape(1, D), Wr[i])
        h, alive3, f = _k2_call(n_in, k)(
            om, stats, alive3, bn_g[i].reshape(1, D), bn_b[i].reshape(1, D),
            pool_w[i].reshape(D, 1))
        feats.append(f)
        n_in = k
    z = jnp.concatenate(feats, axis=1)
    out = _mlp_call(
        z, W1, b1.reshape(1, D), g1.reshape(1, D), beta1.reshape(1, D),
        W2, b2.reshape(1, D // 2), g2.reshape(1, D // 2),
        beta2.reshape(1, D // 2), W3, b3.reshape(1, 1))
    return out.reshape(G)


# speedup vs baseline: 1.1169x; 1.1169x over previous
import functools
import math

import jax
import jax.numpy as jnp
from jax import lax
from jax.experimental import pallas as pl
from jax.experimental.pallas import tpu as pltpu
from jax.experimental.pallas import tpu_sc as plsc

N_NODES = 10000
N_EDGES = 320000
G = 200
S = N_NODES // G
D = 128
NUM_LAYERS = 3
RATIO = 0.8

NC, NS = 2, 16
NW = NC * NS
EPT = N_EDGES // NW
ECH = 80
NCH = EPT // ECH
N_PAD = 10240
ROWS_A = N_PAD // NS
ZROWS = 160

@functools.cache
def _sc_mesh():
    return plsc.VectorSubcoreMesh(core_axis_name="c", subcore_axis_name="s")


def _wid():
    return lax.axis_index("s") * NC + lax.axis_index("c")


def _embed_body(emb_hbm, xidx_hbm, h0_hbm, idx_v, rows_v, sem):
    w = _wid()
    for t in range(4):
        ch = w + t * NW

        @pl.when(ch < N_NODES // ECH)
        def _():
            base = ch * ECH
            pltpu.sync_copy(xidx_hbm.at[pl.ds(base, ECH)], idx_v)
            pltpu.async_copy(emb_hbm.at[idx_v], rows_v, sem).wait()
            pltpu.sync_copy(rows_v, h0_hbm.at[pl.ds(base, ECH)])


@functools.cache
def _embed_call():
    return pl.kernel(
        _embed_body,
        out_type=jax.ShapeDtypeStruct((N_NODES, D), jnp.float32),
        mesh=_sc_mesh(),
        scratch_types=[
            pltpu.VMEM((ECH,), jnp.int32),
            pltpu.VMEM((ECH, D), jnp.float32),
            pltpu.SemaphoreType.DMA,
        ],
    )


def _edge_agg_body(h_hbm, alive_hbm, src_hbm, dst_hbm, za_hbm, zc_hbm,
                   pagg_hbm, pcnt_hbm,
                   agg_sh, cnt_sh, zbuf_v,
                   src0_v, dst0_v, rows0_v, av0_v,
                   src1_v, dst1_v, rows1_v, av1_v,
                   sga, sga2, sgb, sgb2, ssa, ssa2, ssb, ssb2):
    c = lax.axis_index("c")
    s = lax.axis_index("s")
    w = s * NC + c

    pltpu.sync_copy(za_hbm, zbuf_v)
    for r in range(ROWS_A // ZROWS):
        pltpu.sync_copy(zbuf_v,
                        agg_sh.at[pl.ds(s * ROWS_A + r * ZROWS, ZROWS)])
    pltpu.sync_copy(zc_hbm, cnt_sh.at[pl.ds(s * ROWS_A, ROWS_A)])
    plsc.subcore_barrier()

    def _wait(dst_rows, dst_av, s_r, s_a):
        pltpu.make_async_copy(h_hbm.at[pl.ds(0, ECH)], dst_rows, s_r).wait()
        pltpu.make_async_copy(alive_hbm.at[pl.ds(0, ECH)], dst_av, s_a).wait()

    def _load_idx(i, sv, dv):
        base = w * EPT + i * ECH
        pltpu.sync_copy(src_hbm.at[pl.ds(base, ECH)], sv)
        pltpu.sync_copy(dst_hbm.at[pl.ds(base, ECH)], dv)

    def _gather(sv, rv, av, s_r, s_a):
        pltpu.async_copy(h_hbm.at[sv], rv, s_r)
        pltpu.async_copy(alive_hbm.at[sv], av, s_a)

    def _scatter(rv, av, dv, s_r, s_a):
        pltpu.async_copy(rv, agg_sh.at[dv], s_r, add=True)
        pltpu.async_copy(av, cnt_sh.at[dv], s_a, add=True)

    _load_idx(0, src0_v, dst0_v)
    _gather(src0_v, rows0_v, av0_v, sga, sga2)

    def body(j, carry):
        i0 = 2 * j

        @pl.when(j >= 1)
        def _():
            _wait(rows1_v, av1_v, ssb, ssb2)

        _load_idx(i0 + 1, src1_v, dst1_v)
        _wait(rows0_v, av0_v, sga, sga2)
        _gather(src1_v, rows1_v, av1_v, sgb, sgb2)
        _scatter(rows0_v, av0_v, dst0_v, ssa, ssa2)
        _wait(rows1_v, av1_v, sgb, sgb2)
        _wait(rows0_v, av0_v, ssa, ssa2)

        @pl.when(i0 + 2 < NCH)
        def _():
            _load_idx(i0 + 2, src0_v, dst0_v)
            _gather(src0_v, rows0_v, av0_v, sga, sga2)

        _scatter(rows1_v, av1_v, dst1_v, ssb, ssb2)
        return carry

    lax.fori_loop(0, NCH // 2, body, 0)

    _wait(rows1_v, av1_v, ssb, ssb2)
    _wait(rows0_v, av0_v, sga, sga2)
    _scatter(rows0_v, av0_v, dst0_v, ssa, ssa2)
    _wait(rows0_v, av0_v, ssa, ssa2)
    plsc.subcore_barrier()

    pltpu.sync_copy(agg_sh.at[pl.ds(s * ROWS_A, ROWS_A)],
                    pagg_hbm.at[c, pl.ds(s * ROWS_A, ROWS_A)])
    pltpu.sync_copy(cnt_sh.at[pl.ds(s * ROWS_A, ROWS_A)],
                    pcnt_hbm.at[c, pl.ds(s * ROWS_A, ROWS_A)])


@functools.cache
def _edge_agg_call():
    return pl.kernel(
        _edge_agg_body,
        out_type=[jax.ShapeDtypeStruct((NC, N_PAD, D), jnp.float32),
                  jax.ShapeDtypeStruct((NC, N_PAD), jnp.float32)],
        mesh=_sc_mesh(),
        scratch_types=[
            pltpu.VMEM_SHARED((N_PAD, D), jnp.float32),
            pltpu.VMEM_SHARED((N_PAD,), jnp.float32),
            pltpu.VMEM((ZROWS, D), jnp.float32),
            pltpu.VMEM((ECH,), jnp.int32),
            pltpu.VMEM((ECH,), jnp.int32),
            pltpu.VMEM((ECH, D), jnp.float32),
            pltpu.VMEM((ECH,), jnp.float32),
            pltpu.VMEM((ECH,), jnp.int32),
            pltpu.VMEM((ECH,), jnp.int32),
            pltpu.VMEM((ECH, D), jnp.float32),
            pltpu.VMEM((ECH,), jnp.float32),
        ] + [pltpu.SemaphoreType.DMA] * 8,
    )


NEG = -3e38
BR1 = 2000
GB = 40
BR2 = GB * S


def _tanh(x):
    e = jnp.exp(2.0 * jnp.clip(x, -15.0, 15.0))
    return (e - 1.0) / (e + 1.0)


def _dotd(a, b):
    return jnp.dot(a.astype(jnp.bfloat16), b.astype(jnp.bfloat16),
                   preferred_element_type=jnp.float32)


def _k1_body(pagg_ref, pcnt_ref, h_ref, alive_ref, wl_ref, bl_ref, wr_ref,
             om_ref, stats_ref):
    agg = pagg_ref[0] + pagg_ref[1]
    cnt = pcnt_ref[0] + pcnt_ref[1]
    mean = agg / jnp.maximum(cnt, 1.0)
    out = _dotd(mean, wl_ref[...]) + bl_ref[...] + _dotd(h_ref[...], wr_ref[...])
    nrm = jnp.sqrt(jnp.sum(out * out, axis=1, keepdims=True))
    out = out / jnp.maximum(nrm, 1e-12)
    om = out * alive_ref[...]
    om_ref[...] = om
    st = jnp.concatenate([jnp.sum(om, axis=0, keepdims=True),
                          jnp.sum(om * om, axis=0, keepdims=True)], axis=0)

    @pl.when(pl.program_id(0) == 0)
    def _():
        stats_ref[...] = st

    @pl.when(pl.program_id(0) != 0)
    def _():
        stats_ref[...] += st


def _k1_call():
    nb = N_NODES // BR1
    return pl.pallas_call(
        _k1_body,
        grid=(nb,),
        in_specs=[
            pl.BlockSpec((NC, BR1, D), lambda i: (0, i, 0)),
            pl.BlockSpec((NC, BR1, 1), lambda i: (0, i, 0)),
            pl.BlockSpec((BR1, D), lambda i: (i, 0)),
            pl.BlockSpec((BR1, 1), lambda i: (i, 0)),
            pl.BlockSpec((D, D), lambda i: (0, 0)),
            pl.BlockSpec((1, D), lambda i: (0, 0)),
            pl.BlockSpec((D, D), lambda i: (0, 0)),
        ],
        out_specs=[
            pl.BlockSpec((BR1, D), lambda i: (i, 0)),
            pl.BlockSpec((2, D), lambda i: (0, 0)),
        ],
        out_shape=[jax.ShapeDtypeStruct((N_NODES, D), jnp.float32),
                   jax.ShapeDtypeStruct((2, D), jnp.float32)],
    )


def _k2_body(om_ref, stats_ref, alive_ref, bng_ref, bnb_ref, pw_ref,
             hout_ref, aout_ref, feat_ref, *, n_in, k):
    n_cur = float(G * n_in)
    mu = stats_ref[0:1, :] / n_cur
    var = stats_ref[1:2, :] / n_cur - mu * mu
    alive3 = alive_ref[...]
    alive2 = alive3.reshape(BR2, 1)
    out = (om_ref[...] - mu) / jnp.sqrt(var + 1e-5) * bng_ref[...] + bnb_ref[...]
    out = jnp.maximum(out, 0.0) * alive2

    pw = pw_ref[...]
    wn = jnp.sqrt(jnp.sum(pw * pw))
    score2 = _dotd(out, pw) / wn
    score3 = jnp.where(alive3 > 0, score2.reshape(GB, S, 1), jnp.float32(NEG))

    slot = lax.broadcasted_iota(jnp.int32, (GB, S, 1), 1)
    rank = jnp.zeros((GB, S, 1), jnp.float32)
    for l in range(S):
        sl = score3[:, l:l + 1, :]
        rank = rank + jnp.where(sl > score3, 1.0, 0.0)
        rank = rank + jnp.where((sl == score3) & (l < slot), 1.0, 0.0)
    keep3 = jnp.where((rank < k) & (alive3 > 0), 1.0, 0.0)

    h3 = out.reshape(GB, S, D) * (_tanh(score3) * keep3)
    hout_ref[...] = h3.reshape(BR2, D)
    aout_ref[...] = keep3
    hm = jnp.sum(h3, axis=1) / float(k)
    hx = jnp.max(jnp.where(keep3 > 0, h3, jnp.float32(NEG)), axis=1)
    feat_ref[...] = jnp.concatenate([hm, hx], axis=1)


def _k2_call(n_in, k):
    nb = G // GB
    return pl.pallas_call(
        functools.partial(_k2_body, n_in=n_in, k=k),
        grid=(nb,),
        in_specs=[
            pl.BlockSpec((BR2, D), lambda i: (i, 0)),
            pl.BlockSpec((2, D), lambda i: (0, 0)),
            pl.BlockSpec((GB, S, 1), lambda i: (i, 0, 0)),
            pl.BlockSpec((1, D), lambda i: (0, 0)),
            pl.BlockSpec((1, D), lambda i: (0, 0)),
            pl.BlockSpec((D, 1), lambda i: (0, 0)),
        ],
        out_specs=[
            pl.BlockSpec((BR2, D), lambda i: (i, 0)),
            pl.BlockSpec((GB, S, 1), lambda i: (i, 0, 0)),
            pl.BlockSpec((GB, 2 * D), lambda i: (i, 0)),
        ],
        out_shape=[jax.ShapeDtypeStruct((N_NODES, D), jnp.float32),
                   jax.ShapeDtypeStruct((G, S, 1), jnp.float32),
                   jax.ShapeDtypeStruct((G, 2 * D), jnp.float32)],
    )


def _mlp_body(z_ref, w1_ref, b1_ref, g1_ref, be1_ref, w2_ref, b2_ref,
              g2_ref, be2_ref, w3_ref, b3_ref, o_ref):
    z = jnp.maximum(_dotd(z_ref[...], w1_ref[...]) + b1_ref[...], 0.0)
    mu = jnp.sum(z, axis=0, keepdims=True) / float(G)
    var = jnp.sum((z - mu) ** 2, axis=0, keepdims=True) / float(G)
    z = (z - mu) / jnp.sqrt(var + 1e-5) * g1_ref[...] + be1_ref[...]
    z = jnp.maximum(_dotd(z, w2_ref[...]) + b2_ref[...], 0.0)
    mu = jnp.sum(z, axis=0, keepdims=True) / float(G)
    var = jnp.sum((z - mu) ** 2, axis=0, keepdims=True) / float(G)
    z = (z - mu) / jnp.sqrt(var + 1e-5) * g2_ref[...] + be2_ref[...]
    z = _dotd(z, w3_ref[...]) + b3_ref[...]
    o_ref[...] = 1.0 / (1.0 + jnp.exp(-z))


_mlp_call = pl.pallas_call(
    _mlp_body,
    out_shape=jax.ShapeDtypeStruct((G, 1), jnp.float32),
)


def kernel(x, edge_index, batch, emb, Wl, bl, Wr, bn_g, bn_b, pool_w,
           W1, b1, g1, beta1, W2, b2, g2, beta2, W3, b3):
    del batch
    f32 = jnp.float32
    xidx = x.reshape(-1).astype(jnp.int32)
    src = edge_index[0].astype(jnp.int32)
    dst = edge_index[1].astype(jnp.int32)
    za = jnp.zeros((ZROWS, D), f32)
    zc = jnp.zeros((ROWS_A,), f32)

    h = _embed_call()(emb.astype(f32), xidx)
    alive3 = jnp.ones((G, S, 1), f32)

    n_in = S
    feats = []
    for i in range(NUM_LAYERS):
        k = math.ceil(RATIO * n_in)
        pagg, pcnt = _edge_agg_call()(h, alive3.reshape(N_NODES), src, dst,
                                      za, zc)
        pagg = pagg[:, :N_NODES]
        pcnt = pcnt[:, :N_NODES].reshape(NC, N_NODES, 1)
        om, stats = _k1_call()(pagg, pcnt, h, alive3.reshape(N_NODES, 1),
                               Wl[i], bl[i].reshape(1, D), Wr[i])
        h, alive3, f = _k2_call(n_in, k)(
            om, stats, alive3, bn_g[i].reshape(1, D), bn_b[i].reshape(1, D),
            pool_w[i].reshape(D, 1))
        feats.append(f)
        n_in = k
    z = jnp.concatenate(feats, axis=1)
    out = _mlp_call(
        z, W1, b1.reshape(1, D), g1.reshape(1, D), beta1.reshape(1, D),
        W2, b2.reshape(1, D // 2), g2.reshape(1, D // 2),
        beta2.reshape(1, D // 2), W3, b3.reshape(1, 1))
    return out.reshape(G)


# speedup vs baseline: 1.2425x; 1.1124x over previous
import functools
import math

import jax
import jax.numpy as jnp
from jax import lax
from jax.experimental import pallas as pl
from jax.experimental.pallas import tpu as pltpu
from jax.experimental.pallas import tpu_sc as plsc

N_NODES = 10000
N_EDGES = 320000
G = 200
S = N_NODES // G
D = 128
NUM_LAYERS = 3
RATIO = 0.8

NC, NS = 2, 16
NW = NC * NS
EPT = N_EDGES // NW
ECH = 80
NCH = EPT // ECH
N_PAD = 10240
ROWS_A = N_PAD // NS
ZROWS = 160

@functools.cache
def _sc_mesh():
    return plsc.VectorSubcoreMesh(core_axis_name="c", subcore_axis_name="s")


def _wid():
    return lax.axis_index("s") * NC + lax.axis_index("c")


def _embed_body(emb_hbm, xidx_hbm, h0_hbm, idx_v, rows_v, sem):
    w = _wid()
    for t in range(4):
        ch = w + t * NW

        @pl.when(ch < N_NODES // ECH)
        def _():
            base = ch * ECH
            pltpu.sync_copy(xidx_hbm.at[pl.ds(base, ECH)], idx_v)
            pltpu.async_copy(emb_hbm.at[idx_v], rows_v, sem).wait()
            pltpu.sync_copy(rows_v, h0_hbm.at[pl.ds(base, ECH)])


@functools.cache
def _embed_call():
    return pl.kernel(
        _embed_body,
        out_type=jax.ShapeDtypeStruct((N_NODES, D), jnp.float32),
        mesh=_sc_mesh(),
        scratch_types=[
            pltpu.VMEM((ECH,), jnp.int32),
            pltpu.VMEM((ECH, D), jnp.float32),
            pltpu.SemaphoreType.DMA,
        ],
    )


def _edge_agg_body(h_hbm, alive_hbm, ei_hbm, za_hbm, zc_hbm,
                   pagg_hbm, pcnt_hbm,
                   agg_sh, cnt_sh, zbuf_v,
                   idx0_v, rows0_v, av0_v,
                   idx1_v, rows1_v, av1_v,
                   sga, sga2, sgb, sgb2, ssa, ssa2, ssb, ssb2):
    c = lax.axis_index("c")
    s = lax.axis_index("s")
    w = s * NC + c

    pltpu.sync_copy(za_hbm, zbuf_v)
    for r in range(ROWS_A // ZROWS):
        pltpu.sync_copy(zbuf_v,
                        agg_sh.at[pl.ds(s * ROWS_A + r * ZROWS, ZROWS)])
    pltpu.sync_copy(zc_hbm, cnt_sh.at[pl.ds(s * ROWS_A, ROWS_A)])
    plsc.subcore_barrier()

    def _wait(dst_rows, dst_av, s_r, s_a):
        pltpu.make_async_copy(h_hbm.at[pl.ds(0, ECH)], dst_rows, s_r).wait()
        pltpu.make_async_copy(alive_hbm.at[pl.ds(0, ECH)], dst_av, s_a).wait()

    def _load_idx(i, iv):
        pltpu.sync_copy(ei_hbm.at[w * NCH + i], iv)

    def _gather(iv, rv, av, s_r, s_a):
        pltpu.async_copy(h_hbm.at[iv.at[0]], rv, s_r)
        pltpu.async_copy(alive_hbm.at[iv.at[0]], av, s_a)

    def _scatter(rv, av, iv, s_r, s_a):
        pltpu.async_copy(rv, agg_sh.at[iv.at[1]], s_r, add=True)
        pltpu.async_copy(av, cnt_sh.at[iv.at[1]], s_a, add=True)

    _load_idx(0, idx0_v)
    _gather(idx0_v, rows0_v, av0_v, sga, sga2)

    def body(j, carry):
        i0 = 2 * j

        @pl.when(j >= 1)
        def _():
            _wait(rows1_v, av1_v, ssb, ssb2)

        _load_idx(i0 + 1, idx1_v)
        _wait(rows0_v, av0_v, sga, sga2)
        _gather(idx1_v, rows1_v, av1_v, sgb, sgb2)
        _scatter(rows0_v, av0_v, idx0_v, ssa, ssa2)
        _wait(rows1_v, av1_v, sgb, sgb2)
        _wait(rows0_v, av0_v, ssa, ssa2)

        @pl.when(i0 + 2 < NCH)
        def _():
            _load_idx(i0 + 2, idx0_v)
            _gather(idx0_v, rows0_v, av0_v, sga, sga2)

        _scatter(rows1_v, av1_v, idx1_v, ssb, ssb2)
        return carry

    lax.fori_loop(0, NCH // 2, body, 0)

    _wait(rows1_v, av1_v, ssb, ssb2)
    _wait(rows0_v, av0_v, sga, sga2)
    _scatter(rows0_v, av0_v, idx0_v, ssa, ssa2)
    _wait(rows0_v, av0_v, ssa, ssa2)
    plsc.subcore_barrier()

    pltpu.sync_copy(agg_sh.at[pl.ds(s * ROWS_A, ROWS_A)],
                    pagg_hbm.at[c, pl.ds(s * ROWS_A, ROWS_A)])
    pltpu.sync_copy(cnt_sh.at[pl.ds(s * ROWS_A, ROWS_A)],
                    pcnt_hbm.at[c, pl.ds(s * ROWS_A, ROWS_A)])


@functools.cache
def _edge_agg_call():
    return pl.kernel(
        _edge_agg_body,
        out_type=[jax.ShapeDtypeStruct((NC, N_PAD, D), jnp.float32),
                  jax.ShapeDtypeStruct((NC, N_PAD), jnp.float32)],
        mesh=_sc_mesh(),
        scratch_types=[
            pltpu.VMEM_SHARED((N_PAD, D), jnp.float32),
            pltpu.VMEM_SHARED((N_PAD,), jnp.float32),
            pltpu.VMEM((ZROWS, D), jnp.float32),
            pltpu.VMEM((2, ECH), jnp.int32),
            pltpu.VMEM((ECH, D), jnp.float32),
            pltpu.VMEM((ECH,), jnp.float32),
            pltpu.VMEM((2, ECH), jnp.int32),
            pltpu.VMEM((ECH, D), jnp.float32),
            pltpu.VMEM((ECH,), jnp.float32),
        ] + [pltpu.SemaphoreType.DMA] * 8,
    )


NEG = -3e38
BR1 = 2000
GB = 40
BR2 = GB * S


def _tanh(x):
    e = jnp.exp(2.0 * jnp.clip(x, -15.0, 15.0))
    return (e - 1.0) / (e + 1.0)


def _dotd(a, b):
    return jnp.dot(a.astype(jnp.bfloat16), b.astype(jnp.bfloat16),
                   preferred_element_type=jnp.float32)


def _k1_body(pagg_ref, pcnt_ref, h_ref, alive_ref, wl_ref, bl_ref, wr_ref,
             om_ref, stats_ref):
    agg = pagg_ref[0] + pagg_ref[1]
    cnt = pcnt_ref[0] + pcnt_ref[1]
    mean = agg / jnp.maximum(cnt, 1.0)
    out = _dotd(mean, wl_ref[...]) + bl_ref[...] + _dotd(h_ref[...], wr_ref[...])
    nrm = jnp.sqrt(jnp.sum(out * out, axis=1, keepdims=True))
    out = out / jnp.maximum(nrm, 1e-12)
    om = out * alive_ref[...]
    om_ref[...] = om
    st = jnp.concatenate([jnp.sum(om, axis=0, keepdims=True),
                          jnp.sum(om * om, axis=0, keepdims=True)], axis=0)

    @pl.when(pl.program_id(0) == 0)
    def _():
        stats_ref[...] = st

    @pl.when(pl.program_id(0) != 0)
    def _():
        stats_ref[...] += st


def _k1_call():
    nb = N_NODES // BR1
    return pl.pallas_call(
        _k1_body,
        grid=(nb,),
        in_specs=[
            pl.BlockSpec((NC, BR1, D), lambda i: (0, i, 0)),
            pl.BlockSpec((NC, BR1, 1), lambda i: (0, i, 0)),
            pl.BlockSpec((BR1, D), lambda i: (i, 0)),
            pl.BlockSpec((BR1, 1), lambda i: (i, 0)),
            pl.BlockSpec((D, D), lambda i: (0, 0)),
            pl.BlockSpec((1, D), lambda i: (0, 0)),
            pl.BlockSpec((D, D), lambda i: (0, 0)),
        ],
        out_specs=[
            pl.BlockSpec((BR1, D), lambda i: (i, 0)),
            pl.BlockSpec((2, D), lambda i: (0, 0)),
        ],
        out_shape=[jax.ShapeDtypeStruct((N_NODES, D), jnp.float32),
                   jax.ShapeDtypeStruct((2, D), jnp.float32)],
    )


def _k2_body(om_ref, stats_ref, alive_ref, bng_ref, bnb_ref, pw_ref,
             hout_ref, aout_ref, feat_ref, *, n_in, k):
    n_cur = float(G * n_in)
    mu = stats_ref[0:1, :] / n_cur
    var = stats_ref[1:2, :] / n_cur - mu * mu
    alive3 = alive_ref[...]
    alive2 = alive3.reshape(BR2, 1)
    out = (om_ref[...] - mu) / jnp.sqrt(var + 1e-5) * bng_ref[...] + bnb_ref[...]
    out = jnp.maximum(out, 0.0) * alive2

    pw = pw_ref[...]
    wn = jnp.sqrt(jnp.sum(pw * pw))
    score2 = _dotd(out, pw) / wn
    score3 = jnp.where(alive3 > 0, score2.reshape(GB, S, 1), jnp.float32(NEG))

    slot = lax.broadcasted_iota(jnp.int32, (GB, S, 1), 1)
    rank = jnp.zeros((GB, S, 1), jnp.float32)
    for l in range(S):
        sl = score3[:, l:l + 1, :]
        rank = rank + jnp.where(sl > score3, 1.0, 0.0)
        rank = rank + jnp.where((sl == score3) & (l < slot), 1.0, 0.0)
    keep3 = jnp.where((rank < k) & (alive3 > 0), 1.0, 0.0)

    h3 = out.reshape(GB, S, D) * (_tanh(score3) * keep3)
    hout_ref[...] = h3.reshape(BR2, D)
    aout_ref[...] = keep3
    hm = jnp.sum(h3, axis=1) / float(k)
    hx = jnp.max(jnp.where(keep3 > 0, h3, jnp.float32(NEG)), axis=1)
    feat_ref[...] = jnp.concatenate([hm, hx], axis=1)


def _k2_call(n_in, k):
    nb = G // GB
    return pl.pallas_call(
        functools.partial(_k2_body, n_in=n_in, k=k),
        grid=(nb,),
        in_specs=[
            pl.BlockSpec((BR2, D), lambda i: (i, 0)),
            pl.BlockSpec((2, D), lambda i: (0, 0)),
            pl.BlockSpec((GB, S, 1), lambda i: (i, 0, 0)),
            pl.BlockSpec((1, D), lambda i: (0, 0)),
            pl.BlockSpec((1, D), lambda i: (0, 0)),
            pl.BlockSpec((D, 1), lambda i: (0, 0)),
        ],
        out_specs=[
            pl.BlockSpec((BR2, D), lambda i: (i, 0)),
            pl.BlockSpec((GB, S, 1), lambda i: (i, 0, 0)),
            pl.BlockSpec((GB, 2 * D), lambda i: (i, 0)),
        ],
        out_shape=[jax.ShapeDtypeStruct((N_NODES, D), jnp.float32),
                   jax.ShapeDtypeStruct((G, S, 1), jnp.float32),
                   jax.ShapeDtypeStruct((G, 2 * D), jnp.float32)],
    )


def _mlp_body(z_ref, w1_ref, b1_ref, g1_ref, be1_ref, w2_ref, b2_ref,
              g2_ref, be2_ref, w3_ref, b3_ref, o_ref):
    z = jnp.maximum(_dotd(z_ref[...], w1_ref[...]) + b1_ref[...], 0.0)
    mu = jnp.sum(z, axis=0, keepdims=True) / float(G)
    var = jnp.sum((z - mu) ** 2, axis=0, keepdims=True) / float(G)
    z = (z - mu) / jnp.sqrt(var + 1e-5) * g1_ref[...] + be1_ref[...]
    z = jnp.maximum(_dotd(z, w2_ref[...]) + b2_ref[...], 0.0)
    mu = jnp.sum(z, axis=0, keepdims=True) / float(G)
    var = jnp.sum((z - mu) ** 2, axis=0, keepdims=True) / float(G)
    z = (z - mu) / jnp.sqrt(var + 1e-5) * g2_ref[...] + be2_ref[...]
    z = _dotd(z, w3_ref[...]) + b3_ref[...]
    o_ref[...] = 1.0 / (1.0 + jnp.exp(-z))


_mlp_call = pl.pallas_call(
    _mlp_body,
    out_shape=jax.ShapeDtypeStruct((G, 1), jnp.float32),
)


def kernel(x, edge_index, batch, emb, Wl, bl, Wr, bn_g, bn_b, pool_w,
           W1, b1, g1, beta1, W2, b2, g2, beta2, W3, b3):
    del batch
    f32 = jnp.float32
    xidx = x.reshape(-1).astype(jnp.int32)
    ei3 = (edge_index.astype(jnp.int32)
           .reshape(2, N_EDGES // ECH, ECH).transpose(1, 0, 2))
    za = jnp.zeros((ZROWS, D), f32)
    zc = jnp.zeros((ROWS_A,), f32)

    h = _embed_call()(emb.astype(f32), xidx)
    alive3 = jnp.ones((G, S, 1), f32)

    n_in = S
    feats = []
    for i in range(NUM_LAYERS):
        k = math.ceil(RATIO * n_in)
        pagg, pcnt = _edge_agg_call()(h, alive3.reshape(N_NODES), ei3,
                                      za, zc)
        pagg = pagg[:, :N_NODES]
        pcnt = pcnt[:, :N_NODES].reshape(NC, N_NODES, 1)
        om, stats = _k1_call()(pagg, pcnt, h, alive3.reshape(N_NODES, 1),
                               Wl[i], bl[i].reshape(1, D), Wr[i])
        h, alive3, f = _k2_call(n_in, k)(
            om, stats, alive3, bn_g[i].reshape(1, D), bn_b[i].reshape(1, D),
            pool_w[i].reshape(D, 1))
        feats.append(f)
        n_in = k
    z = jnp.concatenate(feats, axis=1)
    out = _mlp_call(
        z, W1, b1.reshape(1, D), g1.reshape(1, D), beta1.reshape(1, D),
        W2, b2.reshape(1, D // 2), g2.reshape(1, D // 2),
        beta2.reshape(1, D // 2), W3, b3.reshape(1, 1))
    return out.reshape(G)
